# baseline (speedup 1.0000x reference)
import functools

import jax
import jax.numpy as jnp
from jax import lax
from jax.experimental import pallas as pl
from jax.experimental.pallas import tpu as pltpu
from jax.experimental.pallas import tpu_sc as plsc

VOCAB_SIZE = 16 * 1024 * 1024
BN_EPS = 1e-5

_NC, _NS, _L = 2, 16, 16
_NW = _NC * _NS
_GW = 128
_NBUF = 7


def _sc_gather_elems(flat_table, idx, d_e):
    n = idx.shape[0]
    per_w = n // _NW
    n_chunks = per_w // _GW
    out_per_chunk = _GW * d_e
    assert per_w % _GW == 0 and n % _NW == 0 and n_chunks % _NBUF == 0
    n_iters = n_chunks // _NBUF
    mesh = plsc.VectorSubcoreMesh(core_axis_name="c", subcore_axis_name="s")

    @functools.partial(
        pl.kernel,
        mesh=mesh,
        compiler_params=pltpu.CompilerParams(use_tc_tiling_on_sc=False,
                                             needs_layout_passes=False),
        out_type=jax.ShapeDtypeStruct((n * d_e,), jnp.float32),
        scratch_types=[
            pltpu.VMEM((per_w,), jnp.int32),
            pltpu.VMEM((_NBUF * out_per_chunk,), jnp.int32),
            pltpu.VMEM((_NBUF * out_per_chunk,), jnp.float32),
            pltpu.SemaphoreType.DMA((_NBUF,)),
            pltpu.SemaphoreType.DMA((_NBUF,)),
        ],
    )
    def gather_kernel(tab_hbm, idx_hbm, out_hbm, idx_v, eidx_v, rows_v,
                      gsem, osem):
        wid = lax.axis_index("s") * _NC + lax.axis_index("c")
        base = wid * per_w
        obase = wid * (per_w * d_e)
        pltpu.sync_copy(idx_hbm.at[pl.ds(base, per_w)], idx_v)

        @pl.loop(0, per_w // _L)
        def _(i):
            r = idx_v[pl.ds(i * _L, _L)]
            idx_v[pl.ds(i * _L, _L)] = ((r >> 7) << 10) + (r & 127)

        lane = lax.iota(jnp.int32, _L)
        sub = lane // d_e
        off = (lane % d_e) * _GW

        def build_eidx(c, j):
            @pl.loop(0, (_GW * d_e) // _L)
            def _(t):
                nvec = c * _GW + t * 2 + sub
                b16 = plsc.load_gather(idx_v, [nvec])
                eidx_v[pl.ds(j * out_per_chunk + t * _L, _L)] = b16 + off

        def fire_gathers(j):
            for p in range(d_e):
                s = j * out_per_chunk + p * _GW
                pltpu.async_copy(
                    tab_hbm.at[eidx_v.at[pl.ds(s, _GW)]],
                    rows_v.at[pl.ds(s, _GW)], gsem.at[j])

        def drain_gathers(j):
            for p in range(d_e):
                pltpu.make_async_copy(
                    out_hbm.at[pl.ds(0, _GW)],
                    rows_v.at[pl.ds(j * out_per_chunk, _GW)],
                    gsem.at[j]).wait()

        def fire_out(c, j):
            pltpu.async_copy(
                rows_v.at[pl.ds(j * out_per_chunk, out_per_chunk)],
                out_hbm.at[pl.ds(obase + c * out_per_chunk, out_per_chunk)],
                osem.at[j])

        def drain_out(j):
            pltpu.make_async_copy(
                rows_v.at[pl.ds(j * out_per_chunk, out_per_chunk)],
                out_hbm.at[pl.ds(0, out_per_chunk)], osem.at[j]).wait()

        for j in range(_NBUF):
            build_eidx(j, j)
            fire_gathers(j)

        @pl.loop(0, n_iters - 1)
        def _(t):
            cbase = t * _NBUF
            for j in range(_NBUF):
                drain_gathers(j)
                fire_out(cbase + j, j)
            for j in range(_NBUF):
                drain_out(j)
                build_eidx(cbase + _NBUF + j, j)
                fire_gathers(j)

        last = (n_iters - 1) * _NBUF
        for j in range(_NBUF):
            drain_gathers(j)
            fire_out(last + j, j)
        for j in range(_NBUF):
            drain_out(j)

    return gather_kernel(flat_table, idx)


def _split_hi_lo(v):
    hi = v.astype(jnp.bfloat16).astype(jnp.float32)
    return hi, v - hi


def _dot3(a, b):
    return jnp.dot(a, b, preferred_element_type=jnp.float32,
                   precision=lax.Precision.HIGHEST)


def _mlp_head(x, state2, counts2, state_table, counts_table, W1, b1, bk):
    b_total, d_x = x.shape
    nb = b_total // bk
    n_state = state_table.shape[0]
    n_counts = counts_table.shape[0]
    d_out = W1.shape[1]

    def body(state_ref, counts_ref, x_ref, st_ref, ct_ref, W1_ref, b1_ref,
             h1_ref, stats_ref, esh_ref, esl_ref, ech_ref, ecl_ref):
        i = pl.program_id(0)

        @pl.when(i == 0)
        def _():
            eff_s = jnp.dot(st_ref[...], W1_ref[0:32, :],
                            preferred_element_type=jnp.float32,
                            precision=lax.Precision.HIGHEST)
            eff_c = jnp.dot(ct_ref[...], W1_ref[32:48, :],
                            preferred_element_type=jnp.float32,
                            precision=lax.Precision.HIGHEST)
            esh_ref[...], esl_ref[...] = _split_hi_lo(eff_s)
            ech_ref[...], ecl_ref[...] = _split_hi_lo(eff_c)
            stats_ref[...] = jnp.zeros_like(stats_ref)

        s_oh = (state_ref[...] ==
                lax.broadcasted_iota(jnp.int32, (bk, n_state), 1)
                ).astype(jnp.float32)
        c_oh = (counts_ref[...] ==
                lax.broadcasted_iota(jnp.int32, (bk, n_counts), 1)
                ).astype(jnp.float32)
        h1 = (_dot3(x_ref[...], W1_ref[48:, :])
              + jnp.dot(s_oh, esh_ref[...], preferred_element_type=jnp.float32)
              + jnp.dot(s_oh, esl_ref[...], preferred_element_type=jnp.float32)
              + jnp.dot(c_oh, ech_ref[...], preferred_element_type=jnp.float32)
              + jnp.dot(c_oh, ecl_ref[...], preferred_element_type=jnp.float32)
              + b1_ref[...])
        h1_ref[...] = h1
        stats_ref[0:1, :] += jnp.sum(h1, axis=0, keepdims=True)
        stats_ref[1:2, :] += jnp.sum(h1 * h1, axis=0, keepdims=True)

    return pl.pallas_call(
        body,
        grid=(nb,),
        in_specs=[
            pl.BlockSpec((bk, 1), lambda i: (i, 0)),
            pl.BlockSpec((bk, 1), lambda i: (i, 0)),
            pl.BlockSpec((bk, d_x), lambda i: (i, 0)),
            pl.BlockSpec(state_table.shape, lambda i: (0, 0)),
            pl.BlockSpec(counts_table.shape, lambda i: (0, 0)),
            pl.BlockSpec(W1.shape, lambda i: (0, 0)),
            pl.BlockSpec((1, d_out), lambda i: (0, 0)),
        ],
        out_specs=[
            pl.BlockSpec((bk, d_out), lambda i: (i, 0)),
            pl.BlockSpec((2, d_out), lambda i: (0, 0)),
        ],
        out_shape=[
            jax.ShapeDtypeStruct((b_total, d_out), jnp.float32),
            jax.ShapeDtypeStruct((2, d_out), jnp.float32),
        ],
        scratch_shapes=[
            pltpu.VMEM((n_state, d_out), jnp.float32),
            pltpu.VMEM((n_state, d_out), jnp.float32),
            pltpu.VMEM((n_counts, d_out), jnp.float32),
            pltpu.VMEM((n_counts, d_out), jnp.float32),
        ],
    )(state2, counts2, x, state_table, counts_table, W1, b1)


def _mlp_tail(h1, stats, inv_b, gamma, beta, W2, b2, W3, b3, W4r, b4, bk):
    b_total, d = h1.shape
    nb = b_total // bk

    def body(h1_ref, stats_ref, gamma_ref, beta_ref, W2_ref, b2_ref,
             W3_ref, b3_ref, W4r_ref, b4_ref, out_ref):
        n_sl = stats_ref.shape[0] // 2
        mean = jnp.zeros((1, d), jnp.float32)
        ex2 = jnp.zeros((1, d), jnp.float32)
        for s in range(n_sl):
            mean = mean + stats_ref[2 * s:2 * s + 1, :]
            ex2 = ex2 + stats_ref[2 * s + 1:2 * s + 2, :]
        mean = mean * inv_b
        ex2 = ex2 * inv_b
        var = ex2 - mean * mean
        a = gamma_ref[...] * lax.rsqrt(var + BN_EPS)
        hn = (h1_ref[...] - mean) * a + beta_ref[...]
        h2 = jnp.maximum(_dot3(hn, W2_ref[...]) + b2_ref[...], 0.0)
        h3 = jnp.maximum(_dot3(h2, W3_ref[...]) + b3_ref[...], 0.0)
        out_ref[...] = (jnp.sum(h3 * W4r_ref[...], axis=1, keepdims=True)
                        + b4_ref[...])

    return pl.pallas_call(
        body,
        grid=(nb,),
        in_specs=[
            pl.BlockSpec((bk, d), lambda i: (i, 0)),
            pl.BlockSpec(stats.shape, lambda i: (0, 0)),
            pl.BlockSpec((1, d), lambda i: (0, 0)),
            pl.BlockSpec((1, d), lambda i: (0, 0)),
            pl.BlockSpec(W2.shape, lambda i: (0, 0)),
            pl.BlockSpec((1, W2.shape[1]), lambda i: (0, 0)),
            pl.BlockSpec(W3.shape, lambda i: (0, 0)),
            pl.BlockSpec((1, W3.shape[1]), lambda i: (0, 0)),
            pl.BlockSpec(W4r.shape, lambda i: (0, 0)),
            pl.BlockSpec((1, 1), lambda i: (0, 0)),
        ],
        out_specs=pl.BlockSpec((bk, 1), lambda i: (i, 0)),
        out_shape=jax.ShapeDtypeStruct((b_total, 1), jnp.float32),
    )(h1, stats, gamma, beta, W2, b2, W3, b3, W4r, b4)


def kernel(state, counts, mp, embed_table, state_table, counts_table,
           W1, b1, gamma, beta, W2, b2, W3, b3, W4, b4):
    b = state.shape[0]
    n_pos = mp.shape[1]
    d_e = embed_table.shape[1]
    idx = (mp.astype(jnp.int32) & (VOCAB_SIZE - 1)).reshape(-1)

    flat_table = embed_table.reshape(VOCAB_SIZE // 128, 128, d_e
                                     ).transpose(0, 2, 1).reshape(-1)

    x = _sc_gather_elems(flat_table, idx, d_e).reshape(b, n_pos * d_e)

    state2 = state.astype(jnp.int32).reshape(b, 1)
    counts2 = counts.astype(jnp.int32).reshape(b, 1)
    h1, stats = _mlp_head(x, state2, counts2, state_table, counts_table,
                          W1, b1.reshape(1, -1), bk=2048)
    out = _mlp_tail(h1, stats, 1.0 / b, gamma.reshape(1, -1),
                    beta.reshape(1, -1), W2, b2.reshape(1, -1),
                    W3, b3.reshape(1, -1), W4.reshape(1, -1),
                    b4.reshape(1, -1), bk=2048)
    return out.reshape(b)


# speedup vs baseline: 1.0801x; 1.0801x over previous
import functools

import jax
import jax.numpy as jnp
from jax import lax
from jax.experimental import pallas as pl
from jax.experimental.pallas import tpu as pltpu
from jax.experimental.pallas import tpu_sc as plsc

VOCAB_SIZE = 16 * 1024 * 1024
BN_EPS = 1e-5

_NC, _NS, _L = 2, 16, 16
_NW = _NC * _NS
_GW = 128
_NBUF = 7


def _sc_gather_elems(flat_table, idx, d_e):
    n = idx.shape[0]
    per_w = n // _NW
    n_chunks = per_w // _GW
    out_per_chunk = _GW * d_e
    assert per_w % _GW == 0 and n % _NW == 0 and n_chunks % _NBUF == 0
    n_iters = n_chunks // _NBUF
    mesh = plsc.VectorSubcoreMesh(core_axis_name="c", subcore_axis_name="s")

    @functools.partial(
        pl.kernel,
        mesh=mesh,
        compiler_params=pltpu.CompilerParams(use_tc_tiling_on_sc=False,
                                             needs_layout_passes=False),
        out_type=jax.ShapeDtypeStruct((n * d_e,), jnp.float32),
        scratch_types=[
            pltpu.VMEM((per_w,), jnp.int32),
            pltpu.VMEM((_NBUF * out_per_chunk,), jnp.int32),
            pltpu.VMEM((_NBUF * out_per_chunk,), jnp.float32),
            pltpu.SemaphoreType.DMA((_NBUF,)),
            pltpu.SemaphoreType.DMA((_NBUF,)),
        ],
    )
    def gather_kernel(tab_hbm, idx_hbm, out_hbm, idx_v, eidx_v, rows_v,
                      gsem, osem):
        wid = lax.axis_index("s") * _NC + lax.axis_index("c")
        base = wid * per_w
        obase = wid * (per_w * d_e)
        pltpu.sync_copy(idx_hbm.at[pl.ds(base, per_w)], idx_v)

        @pl.loop(0, per_w // _L)
        def _(i):
            r = idx_v[pl.ds(i * _L, _L)]
            idx_v[pl.ds(i * _L, _L)] = ((r >> 7) << 10) + (r & 127)

        lane = lax.iota(jnp.int32, _L)
        sub = lane // d_e
        off = (lane % d_e) * _GW

        def build_eidx(c, j):
            @pl.loop(0, (_GW * d_e) // _L)
            def _(t):
                nvec = c * _GW + t * 2 + sub
                b16 = plsc.load_gather(idx_v, [nvec])
                eidx_v[pl.ds(j * out_per_chunk + t * _L, _L)] = b16 + off

        def fire_gathers(j):
            for p in range(d_e):
                s = j * out_per_chunk + p * _GW
                pltpu.async_copy(
                    tab_hbm.at[eidx_v.at[pl.ds(s, _GW)]],
                    rows_v.at[pl.ds(s, _GW)], gsem.at[j])

        def drain_gathers(j):
            for p in range(d_e):
                pltpu.make_async_copy(
                    out_hbm.at[pl.ds(0, _GW)],
                    rows_v.at[pl.ds(j * out_per_chunk, _GW)],
                    gsem.at[j]).wait()

        def fire_out(c, j):
            pltpu.async_copy(
                rows_v.at[pl.ds(j * out_per_chunk, out_per_chunk)],
                out_hbm.at[pl.ds(obase + c * out_per_chunk, out_per_chunk)],
                osem.at[j])

        def drain_out(j):
            pltpu.make_async_copy(
                rows_v.at[pl.ds(j * out_per_chunk, out_per_chunk)],
                out_hbm.at[pl.ds(0, out_per_chunk)], osem.at[j]).wait()

        for j in range(_NBUF):
            build_eidx(j, j)
            fire_gathers(j)

        @pl.loop(0, n_iters - 1)
        def _(t):
            cbase = t * _NBUF
            for j in range(_NBUF):
                drain_gathers(j)
                fire_out(cbase + j, j)
            for j in range(_NBUF):
                drain_out(j)
                build_eidx(cbase + _NBUF + j, j)
                fire_gathers(j)

        last = (n_iters - 1) * _NBUF
        for j in range(_NBUF):
            drain_gathers(j)
            fire_out(last + j, j)
        for j in range(_NBUF):
            drain_out(j)

    return gather_kernel(flat_table, idx)


def _split_hi_lo(v):
    hi = v.astype(jnp.bfloat16).astype(jnp.float32)
    return hi, v - hi


def _dot3(a, b):
    ah = a.astype(jnp.bfloat16)
    al = (a - ah.astype(jnp.float32)).astype(jnp.bfloat16)
    bh = b.astype(jnp.bfloat16)
    bl = (b - bh.astype(jnp.float32)).astype(jnp.bfloat16)
    return (jnp.dot(ah, bh, preferred_element_type=jnp.float32)
            + jnp.dot(al, bh, preferred_element_type=jnp.float32)
            + jnp.dot(ah, bl, preferred_element_type=jnp.float32))


def _mlp_head(x, state2, counts2, state_table, counts_table, W1, b1, bk):
    b_total, d_x = x.shape
    nb = b_total // bk
    n_state = state_table.shape[0]
    n_counts = counts_table.shape[0]
    d_out = W1.shape[1]

    def body(state_ref, counts_ref, x_ref, st_ref, ct_ref, W1_ref, b1_ref,
             h1_ref, stats_ref, esh_ref, esl_ref, ech_ref, ecl_ref):
        i = pl.program_id(0)

        @pl.when(i == 0)
        def _():
            eff_s = jnp.dot(st_ref[...], W1_ref[0:32, :],
                            preferred_element_type=jnp.float32,
                            precision=lax.Precision.HIGHEST)
            eff_c = jnp.dot(ct_ref[...], W1_ref[32:48, :],
                            preferred_element_type=jnp.float32,
                            precision=lax.Precision.HIGHEST)
            esh_ref[...], esl_ref[...] = _split_hi_lo(eff_s)
            ech_ref[...], ecl_ref[...] = _split_hi_lo(eff_c)
            stats_ref[...] = jnp.zeros_like(stats_ref)

        s_oh = (state_ref[...] ==
                lax.broadcasted_iota(jnp.int32, (bk, n_state), 1)
                ).astype(jnp.float32)
        c_oh = (counts_ref[...] ==
                lax.broadcasted_iota(jnp.int32, (bk, n_counts), 1)
                ).astype(jnp.float32)
        h1 = (_dot3(x_ref[...], W1_ref[48:, :])
              + jnp.dot(s_oh.astype(jnp.bfloat16),
                        esh_ref[...].astype(jnp.bfloat16),
                        preferred_element_type=jnp.float32)
              + jnp.dot(s_oh.astype(jnp.bfloat16),
                        esl_ref[...].astype(jnp.bfloat16),
                        preferred_element_type=jnp.float32)
              + jnp.dot(c_oh.astype(jnp.bfloat16),
                        ech_ref[...].astype(jnp.bfloat16),
                        preferred_element_type=jnp.float32)
              + jnp.dot(c_oh.astype(jnp.bfloat16),
                        ecl_ref[...].astype(jnp.bfloat16),
                        preferred_element_type=jnp.float32)
              + b1_ref[...])
        h1_ref[...] = h1
        stats_ref[0:1, :] += jnp.sum(h1, axis=0, keepdims=True)
        stats_ref[1:2, :] += jnp.sum(h1 * h1, axis=0, keepdims=True)

    return pl.pallas_call(
        body,
        grid=(nb,),
        in_specs=[
            pl.BlockSpec((bk, 1), lambda i: (i, 0)),
            pl.BlockSpec((bk, 1), lambda i: (i, 0)),
            pl.BlockSpec((bk, d_x), lambda i: (i, 0)),
            pl.BlockSpec(state_table.shape, lambda i: (0, 0)),
            pl.BlockSpec(counts_table.shape, lambda i: (0, 0)),
            pl.BlockSpec(W1.shape, lambda i: (0, 0)),
            pl.BlockSpec((1, d_out), lambda i: (0, 0)),
        ],
        out_specs=[
            pl.BlockSpec((bk, d_out), lambda i: (i, 0)),
            pl.BlockSpec((2, d_out), lambda i: (0, 0)),
        ],
        out_shape=[
            jax.ShapeDtypeStruct((b_total, d_out), jnp.float32),
            jax.ShapeDtypeStruct((2, d_out), jnp.float32),
        ],
        scratch_shapes=[
            pltpu.VMEM((n_state, d_out), jnp.float32),
            pltpu.VMEM((n_state, d_out), jnp.float32),
            pltpu.VMEM((n_counts, d_out), jnp.float32),
            pltpu.VMEM((n_counts, d_out), jnp.float32),
        ],
    )(state2, counts2, x, state_table, counts_table, W1, b1)


def _mlp_tail(h1, stats, inv_b, gamma, beta, W2, b2, W3, b3, W4r, b4, bk):
    b_total, d = h1.shape
    nb = b_total // bk

    def body(h1_ref, stats_ref, gamma_ref, beta_ref, W2_ref, b2_ref,
             W3_ref, b3_ref, W4r_ref, b4_ref, out_ref):
        n_sl = stats_ref.shape[0] // 2
        mean = jnp.zeros((1, d), jnp.float32)
        ex2 = jnp.zeros((1, d), jnp.float32)
        for s in range(n_sl):
            mean = mean + stats_ref[2 * s:2 * s + 1, :]
            ex2 = ex2 + stats_ref[2 * s + 1:2 * s + 2, :]
        mean = mean * inv_b
        ex2 = ex2 * inv_b
        var = ex2 - mean * mean
        a = gamma_ref[...] * lax.rsqrt(var + BN_EPS)
        hn = (h1_ref[...] - mean) * a + beta_ref[...]
        h2 = jnp.maximum(_dot3(hn, W2_ref[...]) + b2_ref[...], 0.0)
        h3 = jnp.maximum(_dot3(h2, W3_ref[...]) + b3_ref[...], 0.0)
        out_ref[...] = (jnp.sum(h3 * W4r_ref[...], axis=1, keepdims=True)
                        + b4_ref[...])

    return pl.pallas_call(
        body,
        grid=(nb,),
        in_specs=[
            pl.BlockSpec((bk, d), lambda i: (i, 0)),
            pl.BlockSpec(stats.shape, lambda i: (0, 0)),
            pl.BlockSpec((1, d), lambda i: (0, 0)),
            pl.BlockSpec((1, d), lambda i: (0, 0)),
            pl.BlockSpec(W2.shape, lambda i: (0, 0)),
            pl.BlockSpec((1, W2.shape[1]), lambda i: (0, 0)),
            pl.BlockSpec(W3.shape, lambda i: (0, 0)),
            pl.BlockSpec((1, W3.shape[1]), lambda i: (0, 0)),
            pl.BlockSpec(W4r.shape, lambda i: (0, 0)),
            pl.BlockSpec((1, 1), lambda i: (0, 0)),
        ],
        out_specs=pl.BlockSpec((bk, 1), lambda i: (i, 0)),
        out_shape=jax.ShapeDtypeStruct((b_total, 1), jnp.float32),
    )(h1, stats, gamma, beta, W2, b2, W3, b3, W4r, b4)


def kernel(state, counts, mp, embed_table, state_table, counts_table,
           W1, b1, gamma, beta, W2, b2, W3, b3, W4, b4):
    b = state.shape[0]
    n_pos = mp.shape[1]
    d_e = embed_table.shape[1]
    idx = (mp.astype(jnp.int32) & (VOCAB_SIZE - 1)).reshape(-1)

    flat_table = embed_table.reshape(VOCAB_SIZE // 128, 128, d_e
                                     ).transpose(0, 2, 1).reshape(-1)

    x = _sc_gather_elems(flat_table, idx, d_e).reshape(b, n_pos * d_e)

    state2 = state.astype(jnp.int32).reshape(b, 1)
    counts2 = counts.astype(jnp.int32).reshape(b, 1)
    h1, stats = _mlp_head(x, state2, counts2, state_table, counts_table,
                          W1, b1.reshape(1, -1), bk=2048)
    out = _mlp_tail(h1, stats, 1.0 / b, gamma.reshape(1, -1),
                    beta.reshape(1, -1), W2, b2.reshape(1, -1),
                    W3, b3.reshape(1, -1), W4.reshape(1, -1),
                    b4.reshape(1, -1), bk=2048)
    return out.reshape(b)
